# baseline (speedup 1.0000x reference)
import functools

import jax
import jax.numpy as jnp
from jax import lax
from jax.experimental import pallas as pl
from jax.experimental.pallas import tpu as pltpu
from jax.experimental.pallas import tpu_sc as plsc

N = 10000
NPAD = 10240
E = 320000
CH = 96
NTILES = 16
C0PW = 106
C1PW = 105
R0TOT = NTILES * C0PW
LROWS = R0TOT + NTILES * C1PW
EPAD = LROWS * CH
SEG0 = ((0, 40), (40, 40), (80, 26))
SEG1 = ((0, 40), (40, 40), (80, 25))
IB = 40
TPT = NPAD // NTILES
ZR = 32

SELU_ALPHA = 1.6732632423543772
SELU_SCALE = 1.0507009873554805


def _zero_vmem2d(ref, rows, cols):
    z = jnp.zeros((16,), jnp.float32)
    per_row = cols // 16

    def body(i, carry):
        r = i // per_row
        k = i % per_row
        ref[r, pl.ds(k * 16, 16)] = z
        return carry

    lax.fori_loop(0, rows * per_row, body, 0)


def _make_prop(D):
    mesh = plsc.VectorSubcoreMesh(core_axis_name="c", subcore_axis_name="s")

    @functools.partial(
        pl.kernel,
        out_type=jax.ShapeDtypeStruct((2, NPAD, D), jnp.float32),
        mesh=mesh,
        scratch_types=[
            pltpu.VMEM((IB, CH), jnp.int32),
            pltpu.VMEM((IB, CH), jnp.int32),
            pltpu.VMEM((2, CH, D), jnp.float32),
            pltpu.VMEM((ZR, D), jnp.float32),
            pltpu.VMEM_SHARED((NPAD, D), jnp.float32),
            pltpu.SemaphoreType.DMA,
            pltpu.SemaphoreType.DMA,
        ],
        compiler_params=pltpu.CompilerParams(use_tc_tiling_on_sc=False),
    )
    def prop(h_hbm, src_hbm, dst_hbm, out_hbm, src_v, dst_v, rows_v, zb_v, acc,
             gsem, ssem):
        c = lax.axis_index("c")
        s = lax.axis_index("s")
        _zero_vmem2d(zb_v, ZR, D)

        def zbody(z, carry):
            pltpu.sync_copy(zb_v, acc.at[pl.ds(s * TPT + z * ZR, ZR)])
            return carry

        lax.fori_loop(0, TPT // ZR, zbody, 0)
        plsc.subcore_barrier()

        def run(base, segs):
            for off, seg_len in segs:
                pltpu.sync_copy(src_hbm.at[pl.ds(base + off, seg_len)],
                                src_v.at[pl.ds(0, seg_len)])
                pltpu.sync_copy(dst_hbm.at[pl.ds(base + off, seg_len)],
                                dst_v.at[pl.ds(0, seg_len)])
                pltpu.async_copy(h_hbm.at[src_v.at[0]], rows_v.at[0], gsem)

                def ebody(j, carry, seg_len=seg_len):
                    b = lax.rem(j, 2)
                    nb = 1 - b
                    pltpu.make_async_copy(
                        h_hbm.at[src_v.at[j]], rows_v.at[b], gsem
                    ).wait()

                    @pl.when(j > 0)
                    def _():
                        pltpu.make_async_copy(
                            rows_v.at[nb], acc.at[dst_v.at[j - 1]], ssem
                        ).wait()

                    @pl.when(j + 1 < seg_len)
                    def _():
                        pltpu.async_copy(h_hbm.at[src_v.at[j + 1]], rows_v.at[nb],
                                         gsem)

                    pltpu.async_copy(rows_v.at[b], acc.at[dst_v.at[j]], ssem,
                                     add=True)
                    return carry

                lax.fori_loop(0, seg_len, ebody, 0)
                pltpu.make_async_copy(
                    rows_v.at[(seg_len - 1) % 2], acc.at[dst_v.at[seg_len - 1]],
                    ssem
                ).wait()

        @pl.when(c == 0)
        def _():
            run(s * C0PW, SEG0)

        @pl.when(c == 1)
        def _():
            run(R0TOT + s * C1PW, SEG1)

        plsc.subcore_barrier()
        pltpu.sync_copy(acc.at[pl.ds(s * TPT, TPT)], out_hbm.at[c, pl.ds(s * TPT, TPT)])

    return prop


_sc_deg = _make_prop(16)
_sc_prop128 = _make_prop(128)
_sc_prop64 = _make_prop(64)
_sc_prop32 = _make_prop(32)


NB = 256
GRID = NPAD // NB


def _rowspec(d):
    return pl.BlockSpec((NB, d), lambda i: (i, 0))


def _fullspec(shape):
    nd = len(shape)
    return pl.BlockSpec(shape, lambda i: (0,) * nd)


def _tc_prep(degp, xp):
    def kern(degp_ref, x_ref, dinv_ref, dinv2_ref, dcinv_ref, s0_ref):
        deg = degp_ref[0][:, 0:1] + degp_ref[1][:, 0:1]
        dinv = jnp.where(deg > 0, lax.rsqrt(jnp.maximum(deg, 1e-12)), 0.0)
        dinv_ref[...] = dinv
        dinv2_ref[...] = dinv * dinv
        dcinv_ref[...] = 1.0 / jnp.maximum(deg, 1.0)
        s0_ref[...] = x_ref[...] * dinv

    return pl.pallas_call(
        kern,
        grid=(GRID,),
        in_specs=[
            pl.BlockSpec((2, NB, 16), lambda i: (0, i, 0)),
            _rowspec(128),
        ],
        out_specs=[_rowspec(1), _rowspec(1), _rowspec(1), _rowspec(128)],
        out_shape=[
            jax.ShapeDtypeStruct((NPAD, 1), jnp.float32),
            jax.ShapeDtypeStruct((NPAD, 1), jnp.float32),
            jax.ShapeDtypeStruct((NPAD, 1), jnp.float32),
            jax.ShapeDtypeStruct((NPAD, 128), jnp.float32),
        ],
    )(degp, xp)


def _tc_scale(ua, ub, sc):
    d = ua.shape[-1]

    def kern(a_ref, b_ref, s_ref, o_ref):
        o_ref[...] = (a_ref[...] + b_ref[...]) * s_ref[...]

    return pl.pallas_call(
        kern,
        grid=(GRID,),
        in_specs=[_rowspec(d), _rowspec(d), _rowspec(1)],
        out_specs=_rowspec(d),
        out_shape=jax.ShapeDtypeStruct((NPAD, d), jnp.float32),
    )(ua, ub, sc)


def _tc_conv1a(u2a, u2b, dinv, wl, wr):
    def kern(a_ref, b_ref, di_ref, wl_ref, wr_ref, hl_ref, hr_ref):
        h = (a_ref[...] + b_ref[...]) * di_ref[...]
        hl_ref[...] = jnp.dot(h, wl_ref[...], preferred_element_type=jnp.float32)
        hr_ref[...] = jnp.dot(h, wr_ref[...], preferred_element_type=jnp.float32)

    return pl.pallas_call(
        kern,
        grid=(GRID,),
        in_specs=[
            _rowspec(128), _rowspec(128), _rowspec(1),
            _fullspec((128, 64)), _fullspec((128, 64)),
        ],
        out_specs=[_rowspec(64), _rowspec(64)],
        out_shape=[
            jax.ShapeDtypeStruct((NPAD, 64), jnp.float32),
            jax.ShapeDtypeStruct((NPAD, 64), jnp.float32),
        ],
    )(u2a, u2b, dinv, wl, wr)


def _tc_conv1b(v1a, v1b, dcinv, hr, b, wl2, wr2):
    def kern(a_ref, b_ref, dc_ref, hr_ref, bias_ref, wl_ref, wr_ref,
             h1l_ref, h1r_ref):
        z = (a_ref[...] + b_ref[...]) * dc_ref[...] + hr_ref[...] + bias_ref[...]
        h1 = SELU_SCALE * jnp.where(z > 0, z, SELU_ALPHA * (jnp.exp(z) - 1.0))
        h1l_ref[...] = jnp.dot(h1, wl_ref[...], preferred_element_type=jnp.float32)
        h1r_ref[...] = jnp.dot(h1, wr_ref[...], preferred_element_type=jnp.float32)

    return pl.pallas_call(
        kern,
        grid=(GRID,),
        in_specs=[
            _rowspec(64), _rowspec(64), _rowspec(1), _rowspec(64),
            _fullspec((1, 64)), _fullspec((64, 32)), _fullspec((64, 32)),
        ],
        out_specs=[_rowspec(32), _rowspec(32)],
        out_shape=[
            jax.ShapeDtypeStruct((NPAD, 32), jnp.float32),
            jax.ShapeDtypeStruct((NPAD, 32), jnp.float32),
        ],
    )(v1a, v1b, dcinv, hr, b, wl2, wr2)


def _tc_conv2(v2a, v2b, dcinv, h1r, b, t, dinv):
    def kern(a_ref, b_ref, dc_ref, hr_ref, bias_ref, t_ref, di_ref,
             p_ref, pyp_ref, s2_ref):
        z = (a_ref[...] + b_ref[...]) * dc_ref[...] + hr_ref[...] + bias_ref[...]
        m = jnp.max(z, axis=1, keepdims=True)
        e = jnp.exp(z - m)
        p = e / jnp.sum(e, axis=1, keepdims=True)
        p_ref[...] = p
        pyp = jnp.dot(p, t_ref[...], preferred_element_type=jnp.float32)
        pyp_ref[...] = pyp
        s2_ref[...] = pyp * di_ref[...]

    return pl.pallas_call(
        kern,
        grid=(GRID,),
        in_specs=[
            _rowspec(32), _rowspec(32), _rowspec(1), _rowspec(32),
            _fullspec((1, 32)), _fullspec((32, 32)), _rowspec(1),
        ],
        out_specs=[_rowspec(32), _rowspec(32), _rowspec(32)],
        out_shape=[
            jax.ShapeDtypeStruct((NPAD, 32), jnp.float32),
            jax.ShapeDtypeStruct((NPAD, 32), jnp.float32),
            jax.ShapeDtypeStruct((NPAD, 32), jnp.float32),
        ],
    )(v2a, v2b, dcinv, h1r, b, t, dinv)


def kernel(x, edge_index, T, Wl1, Wr1, b1, Wl2, Wr2, b2):
    src = edge_index[0].astype(jnp.int32)
    dst = edge_index[1].astype(jnp.int32)
    pad = EPAD - E
    src2d = jnp.concatenate([src, jnp.zeros((pad,), jnp.int32)]).reshape(LROWS, CH)
    dst_pad = N + (jnp.arange(pad, dtype=jnp.int32) % (NPAD - N))
    dst2d = jnp.concatenate([dst, dst_pad]).reshape(LROWS, CH)
    xp = jnp.pad(x, ((0, NPAD - N), (0, 0)))

    degp = _sc_deg(jnp.ones((NPAD, 16), jnp.float32), src2d, dst2d)
    dinv, dinv2, dcinv, s0 = _tc_prep(degp, xp)

    u1 = _sc_prop128(s0, src2d, dst2d)
    s1 = _tc_scale(u1[0], u1[1], dinv2)
    u2 = _sc_prop128(s1, src2d, dst2d)

    hl, hr = _tc_conv1a(u2[0], u2[1], dinv, Wl1.T, Wr1.T)
    v1 = _sc_prop64(hl, src2d, dst2d)
    h1l, h1r = _tc_conv1b(
        v1[0], v1[1], dcinv, hr, b1.reshape(1, -1), Wl2.T, Wr2.T
    )

    v2 = _sc_prop32(h1l, src2d, dst2d)
    p, pyp, s2 = _tc_conv2(
        v2[0], v2[1], dcinv, h1r, b2.reshape(1, -1), T, dinv
    )

    w1 = _sc_prop32(s2, src2d, dst2d)
    s3 = _tc_scale(w1[0], w1[1], dinv2)
    w2 = _sc_prop32(s3, src2d, dst2d)
    pyt = _tc_scale(w2[0], w2[1], dinv)

    return (p[:N], pyp[:N], pyt[:N])


# speedup vs baseline: 1.0910x; 1.0910x over previous
import functools

import jax
import jax.numpy as jnp
from jax import lax
from jax.experimental import pallas as pl
from jax.experimental.pallas import tpu as pltpu
from jax.experimental.pallas import tpu_sc as plsc

N = 10000
NPAD = 10240
E = 320000
CH = 128
NTILES = 16
C0PW = 90
C1PW = 68
R0TOT = NTILES * C0PW
LROWS = R0TOT + NTILES * C1PW
EPAD = LROWS * CH
SEG0 = ((0, 30), (30, 30), (60, 30))
SEG1 = ((0, 30), (30, 30), (60, 8))
IB = 30
ZR = 32
TPT = NPAD // NTILES

SELU_ALPHA = 1.6732632423543772
SELU_SCALE = 1.0507009873554805


def _zero_vmem2d(ref, rows, cols):
    z = jnp.zeros((16,), jnp.float32)
    per_row = cols // 16

    def body(i, carry):
        r = i // per_row
        k = i % per_row
        ref[r, pl.ds(k * 16, 16)] = z
        return carry

    lax.fori_loop(0, rows * per_row, body, 0)


def _make_prop(D):
    mesh = plsc.VectorSubcoreMesh(core_axis_name="c", subcore_axis_name="s")

    @functools.partial(
        pl.kernel,
        out_type=jax.ShapeDtypeStruct((2, NPAD, D), jnp.float32),
        mesh=mesh,
        scratch_types=[
            pltpu.VMEM((IB, CH), jnp.int32),
            pltpu.VMEM((IB, CH), jnp.int32),
            pltpu.VMEM((2, CH, D), jnp.float32),
            pltpu.VMEM((ZR, D), jnp.float32),
            pltpu.VMEM_SHARED((NPAD, D), jnp.float32),
            pltpu.SemaphoreType.DMA,
            pltpu.SemaphoreType.DMA,
        ],
        compiler_params=pltpu.CompilerParams(use_tc_tiling_on_sc=False),
    )
    def prop(h_hbm, src_hbm, dst_hbm, out_hbm, src_v, dst_v, rows_v, zb_v, acc,
             gsem, ssem):
        c = lax.axis_index("c")
        s = lax.axis_index("s")
        _zero_vmem2d(zb_v, ZR, D)

        def zbody(z, carry):
            pltpu.sync_copy(zb_v, acc.at[pl.ds(s * TPT + z * ZR, ZR)])
            return carry

        lax.fori_loop(0, TPT // ZR, zbody, 0)
        plsc.subcore_barrier()

        def run(base, segs):
            for off, seg_len in segs:
                pltpu.sync_copy(src_hbm.at[pl.ds(base + off, seg_len)],
                                src_v.at[pl.ds(0, seg_len)])
                pltpu.sync_copy(dst_hbm.at[pl.ds(base + off, seg_len)],
                                dst_v.at[pl.ds(0, seg_len)])
                pltpu.async_copy(h_hbm.at[src_v.at[0]], rows_v.at[0], gsem)

                def ebody(j, carry, seg_len=seg_len):
                    b = lax.rem(j, 2)
                    nb = 1 - b
                    pltpu.make_async_copy(
                        h_hbm.at[src_v.at[j]], rows_v.at[b], gsem
                    ).wait()

                    @pl.when(j > 0)
                    def _():
                        pltpu.make_async_copy(
                            rows_v.at[nb], acc.at[dst_v.at[j - 1]], ssem
                        ).wait()

                    @pl.when(j + 1 < seg_len)
                    def _():
                        pltpu.async_copy(h_hbm.at[src_v.at[j + 1]], rows_v.at[nb],
                                         gsem)

                    pltpu.async_copy(rows_v.at[b], acc.at[dst_v.at[j]], ssem,
                                     add=True)
                    return carry

                lax.fori_loop(0, seg_len, ebody, 0)
                pltpu.make_async_copy(
                    rows_v.at[(seg_len - 1) % 2], acc.at[dst_v.at[seg_len - 1]],
                    ssem
                ).wait()

        @pl.when(c == 0)
        def _():
            run(s * C0PW, SEG0)

        @pl.when(c == 1)
        def _():
            run(R0TOT + s * C1PW, SEG1)

        plsc.subcore_barrier()
        pltpu.sync_copy(acc.at[pl.ds(s * TPT, TPT)], out_hbm.at[c, pl.ds(s * TPT, TPT)])

    return prop


_sc_deg = _make_prop(16)
_sc_prop128 = _make_prop(128)
_sc_prop64 = _make_prop(64)
_sc_prop32 = _make_prop(32)


NB = 256
GRID = NPAD // NB


def _rowspec(d):
    return pl.BlockSpec((NB, d), lambda i: (i, 0))


def _fullspec(shape):
    nd = len(shape)
    return pl.BlockSpec(shape, lambda i: (0,) * nd)


def _tc_prep(degp, xp):
    def kern(degp_ref, x_ref, dinv_ref, dinv2_ref, dcinv_ref, s0_ref):
        deg = degp_ref[0][:, 0:1] + degp_ref[1][:, 0:1]
        dinv = jnp.where(deg > 0, lax.rsqrt(jnp.maximum(deg, 1e-12)), 0.0)
        dinv_ref[...] = dinv
        dinv2_ref[...] = dinv * dinv
        dcinv_ref[...] = 1.0 / jnp.maximum(deg, 1.0)
        s0_ref[...] = x_ref[...] * dinv

    return pl.pallas_call(
        kern,
        grid=(GRID,),
        in_specs=[
            pl.BlockSpec((2, NB, 16), lambda i: (0, i, 0)),
            _rowspec(128),
        ],
        out_specs=[_rowspec(1), _rowspec(1), _rowspec(1), _rowspec(128)],
        out_shape=[
            jax.ShapeDtypeStruct((NPAD, 1), jnp.float32),
            jax.ShapeDtypeStruct((NPAD, 1), jnp.float32),
            jax.ShapeDtypeStruct((NPAD, 1), jnp.float32),
            jax.ShapeDtypeStruct((NPAD, 128), jnp.float32),
        ],
    )(degp, xp)


def _tc_scale(ua, ub, sc):
    d = ua.shape[-1]

    def kern(a_ref, b_ref, s_ref, o_ref):
        o_ref[...] = (a_ref[...] + b_ref[...]) * s_ref[...]

    return pl.pallas_call(
        kern,
        grid=(GRID,),
        in_specs=[_rowspec(d), _rowspec(d), _rowspec(1)],
        out_specs=_rowspec(d),
        out_shape=jax.ShapeDtypeStruct((NPAD, d), jnp.float32),
    )(ua, ub, sc)


def _tc_conv1a(u2a, u2b, dinv, wl, wr):
    def kern(a_ref, b_ref, di_ref, wl_ref, wr_ref, hl_ref, hr_ref):
        h = (a_ref[...] + b_ref[...]) * di_ref[...]
        hl_ref[...] = jnp.dot(h, wl_ref[...], preferred_element_type=jnp.float32)
        hr_ref[...] = jnp.dot(h, wr_ref[...], preferred_element_type=jnp.float32)

    return pl.pallas_call(
        kern,
        grid=(GRID,),
        in_specs=[
            _rowspec(128), _rowspec(128), _rowspec(1),
            _fullspec((128, 64)), _fullspec((128, 64)),
        ],
        out_specs=[_rowspec(64), _rowspec(64)],
        out_shape=[
            jax.ShapeDtypeStruct((NPAD, 64), jnp.float32),
            jax.ShapeDtypeStruct((NPAD, 64), jnp.float32),
        ],
    )(u2a, u2b, dinv, wl, wr)


def _tc_conv1b(v1a, v1b, dcinv, hr, b, wl2, wr2):
    def kern(a_ref, b_ref, dc_ref, hr_ref, bias_ref, wl_ref, wr_ref,
             h1l_ref, h1r_ref):
        z = (a_ref[...] + b_ref[...]) * dc_ref[...] + hr_ref[...] + bias_ref[...]
        h1 = SELU_SCALE * jnp.where(z > 0, z, SELU_ALPHA * (jnp.exp(z) - 1.0))
        h1l_ref[...] = jnp.dot(h1, wl_ref[...], preferred_element_type=jnp.float32)
        h1r_ref[...] = jnp.dot(h1, wr_ref[...], preferred_element_type=jnp.float32)

    return pl.pallas_call(
        kern,
        grid=(GRID,),
        in_specs=[
            _rowspec(64), _rowspec(64), _rowspec(1), _rowspec(64),
            _fullspec((1, 64)), _fullspec((64, 32)), _fullspec((64, 32)),
        ],
        out_specs=[_rowspec(32), _rowspec(32)],
        out_shape=[
            jax.ShapeDtypeStruct((NPAD, 32), jnp.float32),
            jax.ShapeDtypeStruct((NPAD, 32), jnp.float32),
        ],
    )(v1a, v1b, dcinv, hr, b, wl2, wr2)


def _tc_conv2(v2a, v2b, dcinv, h1r, b, t, dinv):
    def kern(a_ref, b_ref, dc_ref, hr_ref, bias_ref, t_ref, di_ref,
             p_ref, pyp_ref, s2_ref):
        z = (a_ref[...] + b_ref[...]) * dc_ref[...] + hr_ref[...] + bias_ref[...]
        m = jnp.max(z, axis=1, keepdims=True)
        e = jnp.exp(z - m)
        p = e / jnp.sum(e, axis=1, keepdims=True)
        p_ref[...] = p
        pyp = jnp.dot(p, t_ref[...], preferred_element_type=jnp.float32)
        pyp_ref[...] = pyp
        s2_ref[...] = pyp * di_ref[...]

    return pl.pallas_call(
        kern,
        grid=(GRID,),
        in_specs=[
            _rowspec(32), _rowspec(32), _rowspec(1), _rowspec(32),
            _fullspec((1, 32)), _fullspec((32, 32)), _rowspec(1),
        ],
        out_specs=[_rowspec(32), _rowspec(32), _rowspec(32)],
        out_shape=[
            jax.ShapeDtypeStruct((NPAD, 32), jnp.float32),
            jax.ShapeDtypeStruct((NPAD, 32), jnp.float32),
            jax.ShapeDtypeStruct((NPAD, 32), jnp.float32),
        ],
    )(v2a, v2b, dcinv, h1r, b, t, dinv)


def kernel(x, edge_index, T, Wl1, Wr1, b1, Wl2, Wr2, b2):
    src = edge_index[0].astype(jnp.int32)
    dst = edge_index[1].astype(jnp.int32)
    pad = EPAD - E
    src2d = jnp.concatenate([src, jnp.zeros((pad,), jnp.int32)]).reshape(LROWS, CH)
    dst_pad = N + (jnp.arange(pad, dtype=jnp.int32) % (NPAD - N))
    dst2d = jnp.concatenate([dst, dst_pad]).reshape(LROWS, CH)
    xp = jnp.pad(x, ((0, NPAD - N), (0, 0)))

    degp = _sc_deg(jnp.ones((NPAD, 16), jnp.float32), src2d, dst2d)
    dinv, dinv2, dcinv, s0 = _tc_prep(degp, xp)

    u1 = _sc_prop128(s0, src2d, dst2d)
    s1 = _tc_scale(u1[0], u1[1], dinv2)
    u2 = _sc_prop128(s1, src2d, dst2d)

    hl, hr = _tc_conv1a(u2[0], u2[1], dinv, Wl1.T, Wr1.T)
    v1 = _sc_prop64(hl, src2d, dst2d)
    h1l, h1r = _tc_conv1b(
        v1[0], v1[1], dcinv, hr, b1.reshape(1, -1), Wl2.T, Wr2.T
    )

    v2 = _sc_prop32(h1l, src2d, dst2d)
    p, pyp, s2 = _tc_conv2(
        v2[0], v2[1], dcinv, h1r, b2.reshape(1, -1), T, dinv
    )

    w1 = _sc_prop32(s2, src2d, dst2d)
    s3 = _tc_scale(w1[0], w1[1], dinv2)
    w2 = _sc_prop32(s3, src2d, dst2d)
    pyt = _tc_scale(w2[0], w2[1], dinv)

    return (p[:N], pyp[:N], pyt[:N])
